# baseline (speedup 1.0000x reference)
import functools

import jax
import jax.numpy as jnp
from jax import lax
from jax.experimental import pallas as pl
from jax.experimental.pallas import tpu as pltpu
from jax.experimental.pallas import tpu_sc as plsc

V = 100000
F = 4
D = 64
B = 16384

NC, NS, L = 2, 16, 16
NW = NC * NS
BW = B // NW
CHUNK = 128
NCH = BW // CHUNK

_mesh = plsc.VectorSubcoreMesh(core_axis_name="c", subcore_axis_name="s")


@functools.partial(
    pl.kernel,
    out_type=jax.ShapeDtypeStruct((B, D), jnp.float32),
    mesh=_mesh,
    scratch_types=[
        pltpu.VMEM((F, NCH, CHUNK), jnp.int32),
        pltpu.VMEM((F, NCH, CHUNK), jnp.int32),
        pltpu.VMEM((CHUNK, D), jnp.float32),
        pltpu.VMEM((CHUNK, D), jnp.float32),
        pltpu.VMEM((CHUNK, D), jnp.float32),
        pltpu.VMEM((CHUNK, D), jnp.float32),
        pltpu.VMEM((CHUNK, D), jnp.float32),
        pltpu.VMEM((CHUNK, D), jnp.float32),
        pltpu.VMEM((CHUNK, D), jnp.float32),
        pltpu.VMEM((CHUNK, D), jnp.float32),
        pltpu.VMEM((F, CHUNK), jnp.float32),
        pltpu.VMEM((F, CHUNK), jnp.float32),
        pltpu.VMEM((F * CHUNK,), jnp.float32),
        pltpu.VMEM((CHUNK, D), jnp.float32),
        pltpu.VMEM((CHUNK, D), jnp.float32),
        pltpu.SemaphoreType.DMA,
        pltpu.SemaphoreType.DMA,
        pltpu.SemaphoreType.DMA,
    ],
    compiler_params=pltpu.CompilerParams(needs_layout_passes=False,
                                         use_tc_tiling_on_sc=False),
)
def _eges_sc(idx_hbm, aidx_hbm, tab0_hbm, tab1_hbm, tab2_hbm, tab3_hbm,
             alphat_hbm, out_hbm,
             idx_v, aidx_v, r0a, r1a, r2a, r3a, r0b, r1b, r2b, r3b,
             abufa, abufb, scales, outa, outb, sema, semb, osem):
    wid = lax.axis_index("s") * NC + lax.axis_index("c")
    base = wid * BW

    for f in range(F):
        pltpu.sync_copy(idx_hbm.at[f, wid], idx_v.at[f])
        pltpu.sync_copy(aidx_hbm.at[f, wid], aidx_v.at[f])

    rs = ((r0a, r1a, r2a, r3a), (r0b, r1b, r2b, r3b))
    ab = (abufa, abufb)
    obs = (outa, outb)
    sems = (sema, semb)

    tabs = (tab0_hbm, tab1_hbm, tab2_hbm, tab3_hbm)

    def issue(c, p):
        cps = [pltpu.async_copy(tabs[f].at[idx_v.at[f, c]], rs[p][f], sems[p])
               for f in range(F)]
        cps += [pltpu.async_copy(alphat_hbm.at[aidx_v.at[f, c]],
                                 ab[p].at[f], sems[p])
                for f in range(F)]
        return cps

    pending = {0: issue(0, 0)}
    out_cps = {}

    for c in range(NCH):
        p = c % 2
        if c + 1 < NCH:
            pending[c + 1] = issue(c + 1, (c + 1) % 2)
        for cp in pending.pop(c):
            cp.wait()
        if c - 2 in out_cps:
            out_cps.pop(c - 2).wait()

        a_buf = ab[p]
        r0, r1, r2, r3 = rs[p]
        out_buf = obs[p]

        for g in range(CHUNK // L):
            sl = pl.ds(g * L, L)
            a = [a_buf[f, sl] for f in range(F)]
            w = [jnp.exp(x) for x in a]
            u = [jnp.exp(x) for x in w]
            denom = (u[0] + u[1]) + (u[2] + u[3])
            for f in range(F):
                scales[pl.ds(f * CHUNK + g * L, L)] = w[f] / denom

        def row_body(i, carry2):
            col = jnp.full((L,), i, jnp.int32)
            s0 = plsc.load_gather(scales, [col])
            s1 = plsc.load_gather(scales, [CHUNK + col])
            s2 = plsc.load_gather(scales, [2 * CHUNK + col])
            s3 = plsc.load_gather(scales, [3 * CHUNK + col])
            for j in range(D // L):
                sl = pl.ds(j * L, L)
                out_buf[i, sl] = (r0[i, sl] * s0 + r1[i, sl] * s1
                                  + r2[i, sl] * s2 + r3[i, sl] * s3)
            return carry2

        lax.fori_loop(0, CHUNK, row_body, 0, unroll=4)
        out_cps[c] = pltpu.async_copy(
            out_buf, out_hbm.at[pl.ds(base + c * CHUNK, CHUNK)], osem)

    for c in list(out_cps):
        out_cps.pop(c).wait()


def kernel(inputs, tables, alpha):
    inputs = inputs.astype(jnp.int32)
    foffs = (jnp.arange(F, dtype=jnp.int32) * V)[None, :]
    idx_t = inputs.T.reshape(F, NW, NCH, CHUNK)
    aidx_t = (inputs[:, 0:1] + foffs).T.reshape(F, NW, NCH, CHUNK)
    tabs = [tables[f] for f in range(F)]
    alphat = alpha.T.reshape(F * V)
    return _eges_sc(idx_t, aidx_t, tabs[0], tabs[1], tabs[2], tabs[3], alphat)


# speedup vs baseline: 1.4804x; 1.4804x over previous
import functools

import jax
import jax.numpy as jnp
from jax import lax
from jax.experimental import pallas as pl
from jax.experimental.pallas import tpu as pltpu
from jax.experimental.pallas import tpu_sc as plsc

V = 100000
F = 4
D = 64
B = 16384

NC, NS, L = 2, 16, 16
NW = NC * NS
BW = B // NW
CHUNK = 128
NCH = BW // CHUNK

_mesh = plsc.VectorSubcoreMesh(core_axis_name="c", subcore_axis_name="s")


@functools.partial(
    pl.kernel,
    out_type=jax.ShapeDtypeStruct((B, D), jnp.float32),
    mesh=_mesh,
    scratch_types=[
        pltpu.VMEM((F, NCH, CHUNK), jnp.int32),
        pltpu.VMEM((F, NCH, CHUNK), jnp.int32),
        pltpu.VMEM((CHUNK, D), jnp.float32),
        pltpu.VMEM((CHUNK, D), jnp.float32),
        pltpu.VMEM((CHUNK, D), jnp.float32),
        pltpu.VMEM((CHUNK, D), jnp.float32),
        pltpu.VMEM((CHUNK, D), jnp.float32),
        pltpu.VMEM((CHUNK, D), jnp.float32),
        pltpu.VMEM((CHUNK, D), jnp.float32),
        pltpu.VMEM((CHUNK, D), jnp.float32),
        pltpu.VMEM((F, CHUNK), jnp.float32),
        pltpu.VMEM((F, CHUNK), jnp.float32),
        pltpu.VMEM((F * CHUNK,), jnp.float32),
        pltpu.VMEM((CHUNK, D), jnp.float32),
        pltpu.VMEM((CHUNK, D), jnp.float32),
        pltpu.SemaphoreType.DMA,
        pltpu.SemaphoreType.DMA,
        pltpu.SemaphoreType.DMA,
    ],
    compiler_params=pltpu.CompilerParams(needs_layout_passes=False,
                                         use_tc_tiling_on_sc=False),
)
def _eges_sc(idx_hbm, aidx_hbm, tab_hbm, alphat_hbm, out_hbm,
             idx_v, aidx_v, r0a, r1a, r2a, r3a, r0b, r1b, r2b, r3b,
             abufa, abufb, scales, outa, outb, sema, semb, osem):
    wid = lax.axis_index("s") * NC + lax.axis_index("c")
    base = wid * BW

    for f in range(F):
        pltpu.sync_copy(idx_hbm.at[f, wid], idx_v.at[f])
        pltpu.sync_copy(aidx_hbm.at[f, wid], aidx_v.at[f])

    rs = ((r0a, r1a, r2a, r3a), (r0b, r1b, r2b, r3b))
    ab = (abufa, abufb)
    obs = (outa, outb)
    sems = (sema, semb)

    def issue(c, p):
        cps = [pltpu.async_copy(tab_hbm.at[idx_v.at[f, c]], rs[p][f], sems[p])
               for f in range(F)]
        cps += [pltpu.async_copy(alphat_hbm.at[aidx_v.at[f, c]],
                                 ab[p].at[f], sems[p])
                for f in range(F)]
        return cps

    pending = {0: issue(0, 0)}
    out_cps = {}

    for c in range(NCH):
        p = c % 2
        if c + 1 < NCH:
            pending[c + 1] = issue(c + 1, (c + 1) % 2)
        for cp in pending.pop(c):
            cp.wait()
        if c - 2 in out_cps:
            out_cps.pop(c - 2).wait()

        a_buf = ab[p]
        r0, r1, r2, r3 = rs[p]
        out_buf = obs[p]

        for g in range(CHUNK // L):
            sl = pl.ds(g * L, L)
            a = [a_buf[f, sl] for f in range(F)]
            w = [jnp.exp(x) for x in a]
            u = [jnp.exp(x) for x in w]
            denom = (u[0] + u[1]) + (u[2] + u[3])
            for f in range(F):
                scales[pl.ds(f * CHUNK + g * L, L)] = w[f] / denom

        def row_body(i, carry2):
            col = jnp.full((L,), i, jnp.int32)
            s0 = plsc.load_gather(scales, [col])
            s1 = plsc.load_gather(scales, [CHUNK + col])
            s2 = plsc.load_gather(scales, [2 * CHUNK + col])
            s3 = plsc.load_gather(scales, [3 * CHUNK + col])
            for j in range(D // L):
                sl = pl.ds(j * L, L)
                out_buf[i, sl] = (r0[i, sl] * s0 + r1[i, sl] * s1
                                  + r2[i, sl] * s2 + r3[i, sl] * s3)
            return carry2

        lax.fori_loop(0, CHUNK, row_body, 0, unroll=4)
        out_cps[c] = pltpu.async_copy(
            out_buf, out_hbm.at[pl.ds(base + c * CHUNK, CHUNK)], osem)

    for c in list(out_cps):
        out_cps.pop(c).wait()


def kernel(inputs, tables, alpha):
    inputs = inputs.astype(jnp.int32)
    foffs = (jnp.arange(F, dtype=jnp.int32) * V)[None, :]
    idx_t = (inputs + foffs).T.reshape(F, NW, NCH, CHUNK)
    aidx_t = (inputs[:, 0:1] + foffs).T.reshape(F, NW, NCH, CHUNK)
    tab2d = tables.reshape(F * V, D)
    alphat = alpha.T.reshape(F * V)
    return _eges_sc(idx_t, aidx_t, tab2d, alphat)


# speedup vs baseline: 1.4884x; 1.0054x over previous
import functools

import jax
import jax.numpy as jnp
from jax import lax
from jax.experimental import pallas as pl
from jax.experimental.pallas import tpu as pltpu
from jax.experimental.pallas import tpu_sc as plsc

V = 100000
F = 4
D = 64
B = 16384

NC, NS, L = 2, 16, 16
NW = NC * NS
BW = B // NW
CHUNK = 128
NCH = BW // CHUNK

_mesh = plsc.VectorSubcoreMesh(core_axis_name="c", subcore_axis_name="s")


@functools.partial(
    pl.kernel,
    out_type=jax.ShapeDtypeStruct((B, D), jnp.float32),
    mesh=_mesh,
    scratch_types=[
        pltpu.VMEM((F, NCH, CHUNK), jnp.int32),
        pltpu.VMEM((NCH, CHUNK), jnp.int32),
        pltpu.VMEM((CHUNK, D), jnp.float32),
        pltpu.VMEM((CHUNK, D), jnp.float32),
        pltpu.VMEM((CHUNK, D), jnp.float32),
        pltpu.VMEM((CHUNK, D), jnp.float32),
        pltpu.VMEM((CHUNK, D), jnp.float32),
        pltpu.VMEM((CHUNK, D), jnp.float32),
        pltpu.VMEM((CHUNK, D), jnp.float32),
        pltpu.VMEM((CHUNK, D), jnp.float32),
        pltpu.VMEM((F, CHUNK), jnp.float32),
        pltpu.VMEM((F, CHUNK), jnp.float32),
        pltpu.VMEM((F * CHUNK,), jnp.float32),
        pltpu.VMEM((CHUNK, D), jnp.float32),
        pltpu.VMEM((CHUNK, D), jnp.float32),
        pltpu.SemaphoreType.DMA,
        pltpu.SemaphoreType.DMA,
        pltpu.SemaphoreType.DMA,
    ],
    compiler_params=pltpu.CompilerParams(needs_layout_passes=False,
                                         use_tc_tiling_on_sc=False),
)
def _eges_sc(idx_hbm, aidx_hbm, tab_hbm, alphat_hbm, out_hbm,
             idx_v, aidx_v, r0a, r1a, r2a, r3a, r0b, r1b, r2b, r3b,
             abufa, abufb, scales, outa, outb, sema, semb, osem):
    wid = lax.axis_index("s") * NC + lax.axis_index("c")
    base = wid * BW

    for f in range(F):
        pltpu.sync_copy(idx_hbm.at[f, wid], idx_v.at[f])
    pltpu.sync_copy(aidx_hbm.at[wid], aidx_v)

    rs = ((r0a, r1a, r2a, r3a), (r0b, r1b, r2b, r3b))
    ab = (abufa, abufb)
    obs = (outa, outb)
    sems = (sema, semb)

    def issue(c, p):
        cps = [pltpu.async_copy(tab_hbm.at[idx_v.at[f, c]], rs[p][f], sems[p])
               for f in range(F)]
        cps += [pltpu.async_copy(alphat_hbm.at[f].at[aidx_v.at[c]],
                                 ab[p].at[f], sems[p])
                for f in range(F)]
        return cps

    pending = {0: issue(0, 0)}
    out_cps = {}

    for c in range(NCH):
        p = c % 2
        if c + 1 < NCH:
            pending[c + 1] = issue(c + 1, (c + 1) % 2)
        for cp in pending.pop(c):
            cp.wait()
        if c - 2 in out_cps:
            out_cps.pop(c - 2).wait()

        a_buf = ab[p]
        r0, r1, r2, r3 = rs[p]
        out_buf = obs[p]

        for g in range(CHUNK // L):
            sl = pl.ds(g * L, L)
            a = [a_buf[f, sl] for f in range(F)]
            w = [jnp.exp(x) for x in a]
            u = [jnp.exp(x) for x in w]
            denom = (u[0] + u[1]) + (u[2] + u[3])
            for f in range(F):
                scales[pl.ds(f * CHUNK + g * L, L)] = w[f] / denom

        def row_body(i, carry2):
            col = jnp.full((L,), i, jnp.int32)
            s0 = plsc.load_gather(scales, [col])
            s1 = plsc.load_gather(scales, [CHUNK + col])
            s2 = plsc.load_gather(scales, [2 * CHUNK + col])
            s3 = plsc.load_gather(scales, [3 * CHUNK + col])
            for j in range(D // L):
                sl = pl.ds(j * L, L)
                out_buf[i, sl] = (r0[i, sl] * s0 + r1[i, sl] * s1
                                  + r2[i, sl] * s2 + r3[i, sl] * s3)
            return carry2

        lax.fori_loop(0, CHUNK, row_body, 0, unroll=4)
        out_cps[c] = pltpu.async_copy(
            out_buf, out_hbm.at[pl.ds(base + c * CHUNK, CHUNK)], osem)

    for c in list(out_cps):
        out_cps.pop(c).wait()


def kernel(inputs, tables, alpha):
    inputs = inputs.astype(jnp.int32)
    foffs = (jnp.arange(F, dtype=jnp.int32) * V)[None, :]
    idx_t = (inputs + foffs).T.reshape(F, NW, NCH, CHUNK)
    aidx_t = inputs[:, 0].reshape(NW, NCH, CHUNK)
    tab2d = tables.reshape(F * V, D)
    alphat = alpha.T
    return _eges_sc(idx_t, aidx_t, tab2d, alphat)
